# Initial kernel scaffold; baseline (speedup 1.0000x reference)
import jax
import jax.numpy as jnp
from jax.experimental import pallas as pl


def kernel(x, edge_index, W1, b1, g1, be1, W2, b2, Wg, bg, gn1, bn1, Wsl, bsl, Wsr, gn2, bn2, Wjk, bjk, Wc1, bc1, Wc2, bc2):
    raise NotImplementedError("write your pallas kernel here")


# speedup vs baseline: 13.6716x; 13.6716x over previous
import functools

import jax
import jax.numpy as jnp
from jax import lax
from jax.experimental import pallas as pl
from jax.experimental.pallas import tpu as pltpu
from jax.experimental.pallas import tpu_sc as plsc

_CORES = 2
_SUBCORES = 16
_NW = _CORES * _SUBCORES
_CHUNK = 128


def _ln(x, g, b):
    m = jnp.mean(x, axis=-1, keepdims=True)
    v = jnp.mean((x - m) ** 2, axis=-1, keepdims=True)
    return (x - m) / jnp.sqrt(v + 1e-5) * g + b


def _hist_call(src2d, dst2d, n_pad):
    ch = (src2d.shape[0] * src2d.shape[1]) // (_NW * _CHUNK)
    subrows = n_pad // _SUBCORES
    mesh = plsc.VectorSubcoreMesh(core_axis_name="c", subcore_axis_name="s")

    @functools.partial(
        pl.kernel, mesh=mesh,
        out_type=jax.ShapeDtypeStruct((_CORES, n_pad, 128), jnp.float32),
        scratch_types=[
            pltpu.VMEM((8, _CHUNK), jnp.int32),
            pltpu.VMEM((_CHUNK, 128), jnp.float32),
            pltpu.VMEM_SHARED((n_pad, 128), jnp.float32),
        ])
    def k(src_hbm, dst_hbm, out_hbm, idxv, buf, acc):
        c = lax.axis_index("c")
        s = lax.axis_index("s")
        wid = s * _CORES + c
        one16 = jnp.ones((16,), jnp.float32)
        zero16 = jnp.zeros((16,), jnp.float32)

        def fill_zero(t, carry):
            buf[t // 8, pl.ds((t % 8) * 16, 16)] = zero16
            return carry
        lax.fori_loop(0, _CHUNK * 8, fill_zero, 0)

        base = s * subrows

        def zero_acc(t, carry):
            pltpu.sync_copy(buf, acc.at[pl.ds(base + t * _CHUNK, _CHUNK)])
            return carry
        lax.fori_loop(0, subrows // _CHUNK, zero_acc, 0)

        def fill_lo(t, carry):
            buf[t // 4, pl.ds((t % 4) * 16, 16)] = one16
            return carry
        lax.fori_loop(0, _CHUNK * 4, fill_lo, 0)
        plsc.subcore_barrier()

        def outer_src(jj, carry):
            pltpu.sync_copy(src_hbm.at[pl.ds(wid * ch + jj * 8, 8)], idxv)

            def body(j, carry2):
                pltpu.sync_copy(buf, acc.at[idxv.at[j]], add=True)
                return carry2
            lax.fori_loop(0, 8, body, 0)
            return carry
        lax.fori_loop(0, ch // 8, outer_src, 0)

        def fill_swap_lo(t, carry):
            buf[t // 4, pl.ds((t % 4) * 16, 16)] = zero16
            return carry
        lax.fori_loop(0, _CHUNK * 4, fill_swap_lo, 0)

        def fill_swap_hi(t, carry):
            buf[t // 4, pl.ds(64 + (t % 4) * 16, 16)] = one16
            return carry
        lax.fori_loop(0, _CHUNK * 4, fill_swap_hi, 0)

        def outer_dst(jj, carry):
            pltpu.sync_copy(dst_hbm.at[pl.ds(wid * ch + jj * 8, 8)], idxv)

            def body(j, carry2):
                pltpu.sync_copy(buf, acc.at[idxv.at[j]], add=True)
                return carry2
            lax.fori_loop(0, 8, body, 0)
            return carry
        lax.fori_loop(0, ch // 8, outer_dst, 0)
        plsc.subcore_barrier()

        pltpu.sync_copy(acc.at[pl.ds(base, subrows)],
                        out_hbm.at[c, pl.ds(base, subrows)])

    return k(src2d, dst2d)


def _segsum_call(table, src2d, dst2d):
    n_pad, h = table.shape
    ch = (src2d.shape[0] * src2d.shape[1]) // (_NW * _CHUNK)
    subrows = n_pad // _SUBCORES
    mesh = plsc.VectorSubcoreMesh(core_axis_name="c", subcore_axis_name="s")

    @functools.partial(
        pl.kernel, mesh=mesh,
        out_type=jax.ShapeDtypeStruct((_CORES, n_pad, h), jnp.float32),
        scratch_types=[
            pltpu.VMEM((8, _CHUNK), jnp.int32),
            pltpu.VMEM((8, _CHUNK), jnp.int32),
            pltpu.VMEM((_CHUNK, h), jnp.float32),
            pltpu.VMEM_SHARED((n_pad, h), jnp.float32),
            pltpu.SemaphoreType.DMA,
        ])
    def k(tab_hbm, src_hbm, dst_hbm, out_hbm, sidx, didx, rows, acc, sem):
        c = lax.axis_index("c")
        s = lax.axis_index("s")
        wid = s * _CORES + c
        zero16 = jnp.zeros((16,), jnp.float32)
        lanes = h // 16

        def fill_zero(t, carry):
            rows[t // lanes, pl.ds((t % lanes) * 16, 16)] = zero16
            return carry
        lax.fori_loop(0, _CHUNK * lanes, fill_zero, 0)

        base = s * subrows

        def zero_acc(t, carry):
            pltpu.sync_copy(rows, acc.at[pl.ds(base + t * _CHUNK, _CHUNK)])
            return carry
        lax.fori_loop(0, subrows // _CHUNK, zero_acc, 0)
        plsc.subcore_barrier()

        def outer(jj, carry):
            pltpu.sync_copy(src_hbm.at[pl.ds(wid * ch + jj * 8, 8)], sidx)
            pltpu.sync_copy(dst_hbm.at[pl.ds(wid * ch + jj * 8, 8)], didx)

            def body(j, carry2):
                pltpu.async_copy(tab_hbm.at[sidx.at[j]], rows, sem).wait()
                pltpu.sync_copy(rows, acc.at[didx.at[j]], add=True)
                return carry2
            lax.fori_loop(0, 8, body, 0)
            return carry
        lax.fori_loop(0, ch // 8, outer, 0)
        plsc.subcore_barrier()

        pltpu.sync_copy(acc.at[pl.ds(base, subrows)],
                        out_hbm.at[c, pl.ds(base, subrows)])

    return k(table, src2d, dst2d)


def _stats_call(hist, w1s, n, e, n_pad, h):
    import numpy as _np
    iters = max(int(e).bit_length(), 1)
    kneed = float((n - 1) // 2 + 1)
    mean_c = float(_np.float32(_np.float32(e / n) + _np.float32(1e-6)))

    def body(h_ref, w_ref, sw_ref, aux_ref):
        hsum = h_ref[0] + h_ref[1]
        deg = hsum[:, 0:1]
        indeg = hsum[:, 64:65]
        rows = lax.broadcasted_iota(jnp.int32, (n_pad, 1), 0)
        mask = rows < n

        def bs(i, lohi):
            lo, hi = lohi
            mid = (lo + hi) // 2
            le = jnp.logical_and(mask, deg <= mid.astype(jnp.float32))
            cnt = jnp.sum(jnp.where(le, 1.0, 0.0))
            pred = cnt >= kneed
            return (jnp.where(pred, lo, mid + 1), jnp.where(pred, mid, hi))
        lo, _hi = lax.fori_loop(0, iters, bs, (jnp.int32(0), jnp.int32(e)))
        med = lo.astype(jnp.float32)

        s0 = deg / mean_c
        s1 = jnp.log(1.0 + deg)
        s2 = 1.0 / jnp.sqrt(jnp.maximum(deg, 1.0))
        s3 = (deg > med).astype(jnp.float32)
        w = w_ref[...]
        sw_ref[...] = (s0 * w[0:1, :] + s1 * w[1:2, :]
                       + s2 * w[2:3, :] + s3 * w[3:4, :])
        dis = 1.0 / jnp.sqrt(indeg + 1.0)
        invc = 1.0 / jnp.maximum(indeg, 1.0)
        aux_ref[...] = jnp.concatenate(
            [dis, invc, jnp.zeros((n_pad, h - 2), jnp.float32)], axis=1)

    return pl.pallas_call(
        body,
        out_shape=[jax.ShapeDtypeStruct((n_pad, h), jnp.float32)] * 2,
    )(hist, w1s)


def _node_spec(bn, h):
    return pl.BlockSpec((bn, h), lambda i: (i, 0))


def _full_spec(a):
    return pl.BlockSpec(a.shape, lambda i: tuple(0 for _ in a.shape))


def _dense1_call(x_pad, sw, aux, w1x, b1, g1, be1, w2, b2, wg):
    n_pad, d = x_pad.shape
    h = w2.shape[0]
    bn = n_pad // 8

    def body(x_ref, sw_ref, aux_ref, w1x_ref, b1_ref, g1_ref, be1_ref,
             w2_ref, b2_ref, wg_ref, h1_ref, y_ref):
        xb = x_ref[...]
        nrm = jnp.sqrt(jnp.sum(xb * xb, axis=-1, keepdims=True))
        xn = xb / jnp.maximum(nrm, 1e-12)
        pre = (jnp.dot(xn, w1x_ref[...], preferred_element_type=jnp.float32)
               + sw_ref[...] + b1_ref[...])
        hh = jnp.maximum(_ln(pre, g1_ref[...], be1_ref[...]), 0.0)
        h1 = jnp.dot(hh, w2_ref[...], preferred_element_type=jnp.float32) + b2_ref[...]
        xw = jnp.dot(h1, wg_ref[...], preferred_element_type=jnp.float32)
        h1_ref[...] = h1
        y_ref[...] = xw * aux_ref[:, 0:1]

    return pl.pallas_call(
        body, grid=(n_pad // bn,),
        in_specs=[_node_spec(bn, d), _node_spec(bn, h), _node_spec(bn, h),
                  _full_spec(w1x), _full_spec(b1), _full_spec(g1),
                  _full_spec(be1), _full_spec(w2), _full_spec(b2),
                  _full_spec(wg)],
        out_specs=[_node_spec(bn, h), _node_spec(bn, h)],
        out_shape=[jax.ShapeDtypeStruct((n_pad, h), jnp.float32)] * 2,
    )(x_pad, sw, aux, w1x, b1, g1, be1, w2, b2, wg)


def _dense2_call(h1, y, p1, aux, bg, gn1, bn1, wsr, bsl):
    n_pad, h = h1.shape
    bn = n_pad // 8

    def body(h1_ref, y_ref, p_ref, aux_ref, bg_ref, gn1_ref, bn1_ref,
             wsr_ref, bsl_ref, h2_ref, hr_ref):
        ssum = p_ref[0] + p_ref[1] + y_ref[...]
        agg = ssum * aux_ref[:, 0:1] + bg_ref[...]
        hn = jnp.maximum(_ln(agg, gn1_ref[...], bn1_ref[...]), 0.0)
        h2 = h1_ref[...] + hn
        h2_ref[...] = h2
        hr_ref[...] = (jnp.dot(h2, wsr_ref[...], preferred_element_type=jnp.float32)
                       + bsl_ref[...])

    p_spec = pl.BlockSpec((_CORES, bn, h), lambda i: (0, i, 0))
    return pl.pallas_call(
        body, grid=(n_pad // bn,),
        in_specs=[_node_spec(bn, h), _node_spec(bn, h), p_spec,
                  _node_spec(bn, h), _full_spec(bg), _full_spec(gn1),
                  _full_spec(bn1), _full_spec(wsr), _full_spec(bsl)],
        out_specs=[_node_spec(bn, h), _node_spec(bn, h)],
        out_shape=[jax.ShapeDtypeStruct((n_pad, h), jnp.float32)] * 2,
    )(h1, y, p1, aux, bg, gn1, bn1, wsr, bsl)


def _dense3_call(h1, h2, hr, p2, aux, wsl, gn2, bn2, wjk1, wjk2, wjk3, bjk,
                 wc1, bc1, wc2p, bc2p):
    n_pad, h = h1.shape
    bn = n_pad // 8

    def body(h1_ref, h2_ref, hr_ref, p_ref, aux_ref, wsl_ref, gn2_ref,
             bn2_ref, wjk1_ref, wjk2_ref, wjk3_ref, bjk_ref, wc1_ref,
             bc1_ref, wc2_ref, bc2_ref, out_ref):
        mean = (p_ref[0] + p_ref[1]) * aux_ref[:, 1:2]
        hn = (jnp.dot(mean, wsl_ref[...], preferred_element_type=jnp.float32)
              + hr_ref[...])
        hn = jnp.maximum(_ln(hn, gn2_ref[...], bn2_ref[...]), 0.0)
        h2b = h2_ref[...]
        h3 = h2b + hn
        jk = (jnp.dot(h1_ref[...], wjk1_ref[...], preferred_element_type=jnp.float32)
              + jnp.dot(h2b, wjk2_ref[...], preferred_element_type=jnp.float32)
              + jnp.dot(h3, wjk3_ref[...], preferred_element_type=jnp.float32)
              + bjk_ref[...])
        z = jnp.maximum(
            jnp.dot(jk, wc1_ref[...], preferred_element_type=jnp.float32)
            + bc1_ref[...], 0.0)
        out_ref[...] = (jnp.dot(z, wc2_ref[...], preferred_element_type=jnp.float32)
                        + bc2_ref[...])

    p_spec = pl.BlockSpec((_CORES, bn, h), lambda i: (0, i, 0))
    return pl.pallas_call(
        body, grid=(n_pad // bn,),
        in_specs=[_node_spec(bn, h), _node_spec(bn, h), _node_spec(bn, h),
                  p_spec, _node_spec(bn, h), _full_spec(wsl), _full_spec(gn2),
                  _full_spec(bn2), _full_spec(wjk1), _full_spec(wjk2),
                  _full_spec(wjk3), _full_spec(bjk), _full_spec(wc1),
                  _full_spec(bc1), _full_spec(wc2p), _full_spec(bc2p)],
        out_specs=_node_spec(bn, h),
        out_shape=jax.ShapeDtypeStruct((n_pad, h), jnp.float32),
    )(h1, h2, hr, p2, aux, wsl, gn2, bn2, wjk1, wjk2, wjk3, bjk,
      wc1, bc1, wc2p, bc2p)


def kernel(x, edge_index, W1, b1, g1, be1, W2, b2, Wg, bg, gn1, bn1, Wsl,
           bsl, Wsr, gn2, bn2, Wjk, bjk, Wc1, bc1, Wc2, bc2):
    n, d = x.shape
    h = W2.shape[0]
    c_out = Wc2.shape[1]
    e = edge_index.shape[1]

    n_pad = -(-n // 2560) * 2560
    e_pad = -(-e // (_NW * _CHUNK * 8)) * (_NW * _CHUNK * 8)

    src = edge_index[0]
    dst = edge_index[1]
    pad = e_pad - e
    if pad:
        pad_idx = n + (jnp.arange(pad, dtype=jnp.int32) % (n_pad - n))
        src = jnp.concatenate([src, pad_idx])
        dst = jnp.concatenate([dst, pad_idx])
    src2d = src.reshape(e_pad // _CHUNK, _CHUNK)
    dst2d = dst.reshape(e_pad // _CHUNK, _CHUNK)

    x_pad = jnp.zeros((n_pad, d), jnp.float32).at[:n].set(x)

    w1x = W1[:d]
    w1s = W1[d:]
    row = lambda v: v.reshape(1, -1)
    wjk1, wjk2, wjk3 = Wjk[:h], Wjk[h:2 * h], Wjk[2 * h:]
    wc2p = jnp.zeros((h, h), jnp.float32).at[:, :c_out].set(Wc2)
    bc2p = jnp.zeros((1, h), jnp.float32).at[0, :c_out].set(bc2)

    hist = _hist_call(src2d, dst2d, n_pad)
    sw, aux = _stats_call(hist, w1s, n, e, n_pad, h)
    h1, y = _dense1_call(x_pad, sw, aux, w1x, row(b1), row(g1), row(be1),
                         W2, row(b2), Wg)
    p1 = _segsum_call(y, src2d, dst2d)
    h2, hr = _dense2_call(h1, y, p1, aux, row(bg), row(gn1), row(bn1),
                          Wsr, row(bsl))
    p2 = _segsum_call(h2, src2d, dst2d)
    out = _dense3_call(h1, h2, hr, p2, aux, Wsl, row(gn2), row(bn2),
                       wjk1, wjk2, wjk3, row(bjk), Wc1, row(bc1), wc2p, bc2p)
    return out[:n, :c_out]


# speedup vs baseline: 17.8691x; 1.3070x over previous
import functools

import jax
import jax.numpy as jnp
from jax import lax
from jax.experimental import pallas as pl
from jax.experimental.pallas import tpu as pltpu
from jax.experimental.pallas import tpu_sc as plsc

_CORES = 2
_SUBCORES = 16
_NW = _CORES * _SUBCORES
_CHUNK = 128


def _ln(x, g, b):
    m = jnp.mean(x, axis=-1, keepdims=True)
    v = jnp.mean((x - m) ** 2, axis=-1, keepdims=True)
    return (x - m) / jnp.sqrt(v + 1e-5) * g + b


def _hist_call(src2d, dst2d, n_pad):
    ch = (src2d.shape[0] * src2d.shape[1]) // (_NW * _CHUNK)
    subrows = n_pad // _SUBCORES
    mesh = plsc.VectorSubcoreMesh(core_axis_name="c", subcore_axis_name="s")

    @functools.partial(
        pl.kernel, mesh=mesh,
        out_type=jax.ShapeDtypeStruct((_CORES, n_pad, 128), jnp.float32),
        scratch_types=[
            pltpu.VMEM((8, _CHUNK), jnp.int32),
            pltpu.VMEM((_CHUNK, 128), jnp.float32),
            pltpu.VMEM_SHARED((n_pad, 128), jnp.float32),
        ])
    def k(src_hbm, dst_hbm, out_hbm, idxv, buf, acc):
        c = lax.axis_index("c")
        s = lax.axis_index("s")
        wid = s * _CORES + c
        one16 = jnp.ones((16,), jnp.float32)
        zero16 = jnp.zeros((16,), jnp.float32)

        def fill_zero(t, carry):
            buf[t // 8, pl.ds((t % 8) * 16, 16)] = zero16
            return carry
        lax.fori_loop(0, _CHUNK * 8, fill_zero, 0)

        base = s * subrows

        def zero_acc(t, carry):
            pltpu.sync_copy(buf, acc.at[pl.ds(base + t * _CHUNK, _CHUNK)])
            return carry
        lax.fori_loop(0, subrows // _CHUNK, zero_acc, 0)

        def fill_lo(t, carry):
            buf[t // 4, pl.ds((t % 4) * 16, 16)] = one16
            return carry
        lax.fori_loop(0, _CHUNK * 4, fill_lo, 0)
        plsc.subcore_barrier()

        def outer_src(jj, carry):
            pltpu.sync_copy(src_hbm.at[pl.ds(wid * ch + jj * 8, 8)], idxv)

            def body(j, carry2):
                pltpu.sync_copy(buf, acc.at[idxv.at[j]], add=True)
                return carry2
            lax.fori_loop(0, 8, body, 0)
            return carry
        lax.fori_loop(0, ch // 8, outer_src, 0)

        def fill_swap_lo(t, carry):
            buf[t // 4, pl.ds((t % 4) * 16, 16)] = zero16
            return carry
        lax.fori_loop(0, _CHUNK * 4, fill_swap_lo, 0)

        def fill_swap_hi(t, carry):
            buf[t // 4, pl.ds(64 + (t % 4) * 16, 16)] = one16
            return carry
        lax.fori_loop(0, _CHUNK * 4, fill_swap_hi, 0)

        def outer_dst(jj, carry):
            pltpu.sync_copy(dst_hbm.at[pl.ds(wid * ch + jj * 8, 8)], idxv)

            def body(j, carry2):
                pltpu.sync_copy(buf, acc.at[idxv.at[j]], add=True)
                return carry2
            lax.fori_loop(0, 8, body, 0)
            return carry
        lax.fori_loop(0, ch // 8, outer_dst, 0)
        plsc.subcore_barrier()

        pltpu.sync_copy(acc.at[pl.ds(base, subrows)],
                        out_hbm.at[c, pl.ds(base, subrows)])

    return k(src2d, dst2d)


def _segsum_call(table, src2d, dst2d):
    n_pad, h = table.shape
    ch = (src2d.shape[0] * src2d.shape[1]) // (_NW * _CHUNK)
    subrows = n_pad // _SUBCORES
    mesh = plsc.VectorSubcoreMesh(core_axis_name="c", subcore_axis_name="s")

    @functools.partial(
        pl.kernel, mesh=mesh,
        out_type=jax.ShapeDtypeStruct((_CORES, n_pad, h), jnp.float32),
        scratch_types=[
            pltpu.VMEM((ch // 2, _CHUNK), jnp.int32),
            pltpu.VMEM((ch // 2, _CHUNK), jnp.int32),
            pltpu.VMEM((_CHUNK, h), jnp.float32),
            pltpu.VMEM((_CHUNK, h), jnp.float32),
            pltpu.VMEM_SHARED((n_pad, h), jnp.float32),
            pltpu.SemaphoreType.DMA,
            pltpu.SemaphoreType.DMA,
        ])
    def k(tab_hbm, src_hbm, dst_hbm, out_hbm, sidx, didx, rows_a, rows_b,
          acc, sem_a, sem_b):
        c = lax.axis_index("c")
        s = lax.axis_index("s")
        wid = s * _CORES + c
        zero16 = jnp.zeros((16,), jnp.float32)
        lanes = h // 16

        def fill_zero(t, carry):
            rows_a[t // lanes, pl.ds((t % lanes) * 16, 16)] = zero16
            return carry
        lax.fori_loop(0, _CHUNK * lanes, fill_zero, 0)

        base = s * subrows

        def zero_acc(t, carry):
            pltpu.sync_copy(rows_a, acc.at[pl.ds(base + t * _CHUNK, _CHUNK)])
            return carry
        lax.fori_loop(0, subrows // _CHUNK, zero_acc, 0)
        plsc.subcore_barrier()

        def gat(j, buf, sem):
            return pltpu.make_async_copy(tab_hbm.at[sidx.at[j]], buf, sem)

        def scat(j, buf):
            pltpu.sync_copy(buf, acc.at[didx.at[j]], add=True)

        bch = ch // 2
        for b in range(2):
            pltpu.sync_copy(src_hbm.at[pl.ds(wid * ch + b * bch, bch)], sidx)
            pltpu.sync_copy(dst_hbm.at[pl.ds(wid * ch + b * bch, bch)], didx)
            gat(0, rows_a, sem_a).start()

            def pair(p, carry):
                j = 2 * p
                gat(j + 1, rows_b, sem_b).start()
                gat(j, rows_a, sem_a).wait()
                scat(j, rows_a)
                gat(j + 2, rows_a, sem_a).start()
                gat(j + 1, rows_b, sem_b).wait()
                scat(j + 1, rows_b)
                return carry
            lax.fori_loop(0, bch // 2 - 1, pair, 0)

            je = bch - 2
            gat(je + 1, rows_b, sem_b).start()
            gat(je, rows_a, sem_a).wait()
            scat(je, rows_a)
            gat(je + 1, rows_b, sem_b).wait()
            scat(je + 1, rows_b)
        plsc.subcore_barrier()

        pltpu.sync_copy(acc.at[pl.ds(base, subrows)],
                        out_hbm.at[c, pl.ds(base, subrows)])

    return k(table, src2d, dst2d)


def _stats_call(hist, w1s, n, e, n_pad, h):
    import numpy as _np
    iters = max(int(e).bit_length(), 1)
    kneed = float((n - 1) // 2 + 1)
    mean_c = float(_np.float32(_np.float32(e / n) + _np.float32(1e-6)))

    def body(h_ref, w_ref, sw_ref, aux_ref):
        hsum = h_ref[0] + h_ref[1]
        deg = hsum[:, 0:1]
        indeg = hsum[:, 64:65]
        rows = lax.broadcasted_iota(jnp.int32, (n_pad, 1), 0)
        mask = rows < n

        def bs(i, lohi):
            lo, hi = lohi
            mid = (lo + hi) // 2
            le = jnp.logical_and(mask, deg <= mid.astype(jnp.float32))
            cnt = jnp.sum(jnp.where(le, 1.0, 0.0))
            pred = cnt >= kneed
            return (jnp.where(pred, lo, mid + 1), jnp.where(pred, mid, hi))
        lo, _hi = lax.fori_loop(0, iters, bs, (jnp.int32(0), jnp.int32(e)))
        med = lo.astype(jnp.float32)

        s0 = deg / mean_c
        s1 = jnp.log(1.0 + deg)
        s2 = 1.0 / jnp.sqrt(jnp.maximum(deg, 1.0))
        s3 = (deg > med).astype(jnp.float32)
        w = w_ref[...]
        sw_ref[...] = (s0 * w[0:1, :] + s1 * w[1:2, :]
                       + s2 * w[2:3, :] + s3 * w[3:4, :])
        dis = 1.0 / jnp.sqrt(indeg + 1.0)
        invc = 1.0 / jnp.maximum(indeg, 1.0)
        aux_ref[...] = jnp.concatenate(
            [dis, invc, jnp.zeros((n_pad, h - 2), jnp.float32)], axis=1)

    return pl.pallas_call(
        body,
        out_shape=[jax.ShapeDtypeStruct((n_pad, h), jnp.float32)] * 2,
    )(hist, w1s)


def _node_spec(bn, h):
    return pl.BlockSpec((bn, h), lambda i: (i, 0))


def _full_spec(a):
    return pl.BlockSpec(a.shape, lambda i: tuple(0 for _ in a.shape))


def _dense1_call(x_pad, sw, aux, w1x, b1, g1, be1, w2, b2, wg):
    n_pad, d = x_pad.shape
    h = w2.shape[0]
    bn = n_pad // 8

    def body(x_ref, sw_ref, aux_ref, w1x_ref, b1_ref, g1_ref, be1_ref,
             w2_ref, b2_ref, wg_ref, h1_ref, y_ref):
        xb = x_ref[...]
        nrm = jnp.sqrt(jnp.sum(xb * xb, axis=-1, keepdims=True))
        xn = xb / jnp.maximum(nrm, 1e-12)
        pre = (jnp.dot(xn, w1x_ref[...], preferred_element_type=jnp.float32)
               + sw_ref[...] + b1_ref[...])
        hh = jnp.maximum(_ln(pre, g1_ref[...], be1_ref[...]), 0.0)
        h1 = jnp.dot(hh, w2_ref[...], preferred_element_type=jnp.float32) + b2_ref[...]
        xw = jnp.dot(h1, wg_ref[...], preferred_element_type=jnp.float32)
        h1_ref[...] = h1
        y_ref[...] = xw * aux_ref[:, 0:1]

    return pl.pallas_call(
        body, grid=(n_pad // bn,),
        in_specs=[_node_spec(bn, d), _node_spec(bn, h), _node_spec(bn, h),
                  _full_spec(w1x), _full_spec(b1), _full_spec(g1),
                  _full_spec(be1), _full_spec(w2), _full_spec(b2),
                  _full_spec(wg)],
        out_specs=[_node_spec(bn, h), _node_spec(bn, h)],
        out_shape=[jax.ShapeDtypeStruct((n_pad, h), jnp.float32)] * 2,
    )(x_pad, sw, aux, w1x, b1, g1, be1, w2, b2, wg)


def _dense2_call(h1, y, p1, aux, bg, gn1, bn1, wsr, bsl):
    n_pad, h = h1.shape
    bn = n_pad // 8

    def body(h1_ref, y_ref, p_ref, aux_ref, bg_ref, gn1_ref, bn1_ref,
             wsr_ref, bsl_ref, h2_ref, hr_ref):
        ssum = p_ref[0] + p_ref[1] + y_ref[...]
        agg = ssum * aux_ref[:, 0:1] + bg_ref[...]
        hn = jnp.maximum(_ln(agg, gn1_ref[...], bn1_ref[...]), 0.0)
        h2 = h1_ref[...] + hn
        h2_ref[...] = h2
        hr_ref[...] = (jnp.dot(h2, wsr_ref[...], preferred_element_type=jnp.float32)
                       + bsl_ref[...])

    p_spec = pl.BlockSpec((_CORES, bn, h), lambda i: (0, i, 0))
    return pl.pallas_call(
        body, grid=(n_pad // bn,),
        in_specs=[_node_spec(bn, h), _node_spec(bn, h), p_spec,
                  _node_spec(bn, h), _full_spec(bg), _full_spec(gn1),
                  _full_spec(bn1), _full_spec(wsr), _full_spec(bsl)],
        out_specs=[_node_spec(bn, h), _node_spec(bn, h)],
        out_shape=[jax.ShapeDtypeStruct((n_pad, h), jnp.float32)] * 2,
    )(h1, y, p1, aux, bg, gn1, bn1, wsr, bsl)


def _dense3_call(h1, h2, hr, p2, aux, wsl, gn2, bn2, wjk1, wjk2, wjk3, bjk,
                 wc1, bc1, wc2p, bc2p):
    n_pad, h = h1.shape
    bn = n_pad // 8

    def body(h1_ref, h2_ref, hr_ref, p_ref, aux_ref, wsl_ref, gn2_ref,
             bn2_ref, wjk1_ref, wjk2_ref, wjk3_ref, bjk_ref, wc1_ref,
             bc1_ref, wc2_ref, bc2_ref, out_ref):
        mean = (p_ref[0] + p_ref[1]) * aux_ref[:, 1:2]
        hn = (jnp.dot(mean, wsl_ref[...], preferred_element_type=jnp.float32)
              + hr_ref[...])
        hn = jnp.maximum(_ln(hn, gn2_ref[...], bn2_ref[...]), 0.0)
        h2b = h2_ref[...]
        h3 = h2b + hn
        jk = (jnp.dot(h1_ref[...], wjk1_ref[...], preferred_element_type=jnp.float32)
              + jnp.dot(h2b, wjk2_ref[...], preferred_element_type=jnp.float32)
              + jnp.dot(h3, wjk3_ref[...], preferred_element_type=jnp.float32)
              + bjk_ref[...])
        z = jnp.maximum(
            jnp.dot(jk, wc1_ref[...], preferred_element_type=jnp.float32)
            + bc1_ref[...], 0.0)
        out_ref[...] = (jnp.dot(z, wc2_ref[...], preferred_element_type=jnp.float32)
                        + bc2_ref[...])

    p_spec = pl.BlockSpec((_CORES, bn, h), lambda i: (0, i, 0))
    return pl.pallas_call(
        body, grid=(n_pad // bn,),
        in_specs=[_node_spec(bn, h), _node_spec(bn, h), _node_spec(bn, h),
                  p_spec, _node_spec(bn, h), _full_spec(wsl), _full_spec(gn2),
                  _full_spec(bn2), _full_spec(wjk1), _full_spec(wjk2),
                  _full_spec(wjk3), _full_spec(bjk), _full_spec(wc1),
                  _full_spec(bc1), _full_spec(wc2p), _full_spec(bc2p)],
        out_specs=_node_spec(bn, h),
        out_shape=jax.ShapeDtypeStruct((n_pad, h), jnp.float32),
    )(h1, h2, hr, p2, aux, wsl, gn2, bn2, wjk1, wjk2, wjk3, bjk,
      wc1, bc1, wc2p, bc2p)


def kernel(x, edge_index, W1, b1, g1, be1, W2, b2, Wg, bg, gn1, bn1, Wsl,
           bsl, Wsr, gn2, bn2, Wjk, bjk, Wc1, bc1, Wc2, bc2):
    n, d = x.shape
    h = W2.shape[0]
    c_out = Wc2.shape[1]
    e = edge_index.shape[1]

    n_pad = -(-n // 2560) * 2560
    e_pad = -(-e // (_NW * _CHUNK * 8)) * (_NW * _CHUNK * 8)

    src = edge_index[0]
    dst = edge_index[1]
    pad = e_pad - e
    if pad:
        pad_idx = n + (jnp.arange(pad, dtype=jnp.int32) % (n_pad - n))
        src = jnp.concatenate([src, pad_idx])
        dst = jnp.concatenate([dst, pad_idx])
    src2d = src.reshape(e_pad // _CHUNK, _CHUNK)
    dst2d = dst.reshape(e_pad // _CHUNK, _CHUNK)

    x_pad = jnp.zeros((n_pad, d), jnp.float32).at[:n].set(x)

    w1x = W1[:d]
    w1s = W1[d:]
    row = lambda v: v.reshape(1, -1)
    wjk1, wjk2, wjk3 = Wjk[:h], Wjk[h:2 * h], Wjk[2 * h:]
    wc2p = jnp.zeros((h, h), jnp.float32).at[:, :c_out].set(Wc2)
    bc2p = jnp.zeros((1, h), jnp.float32).at[0, :c_out].set(bc2)

    hist = _hist_call(src2d, dst2d, n_pad)
    sw, aux = _stats_call(hist, w1s, n, e, n_pad, h)
    h1, y = _dense1_call(x_pad, sw, aux, w1x, row(b1), row(g1), row(be1),
                         W2, row(b2), Wg)
    p1 = _segsum_call(y, src2d, dst2d)
    h2, hr = _dense2_call(h1, y, p1, aux, row(bg), row(gn1), row(bn1),
                          Wsr, row(bsl))
    p2 = _segsum_call(h2, src2d, dst2d)
    out = _dense3_call(h1, h2, hr, p2, aux, Wsl, row(gn2), row(bn2),
                       wjk1, wjk2, wjk3, row(bjk), Wc1, row(bc1), wc2p, bc2p)
    return out[:n, :c_out]


# speedup vs baseline: 18.0608x; 1.0107x over previous
import functools

import jax
import jax.numpy as jnp
from jax import lax
from jax.experimental import pallas as pl
from jax.experimental.pallas import tpu as pltpu
from jax.experimental.pallas import tpu_sc as plsc

_CORES = 2
_SUBCORES = 16
_NW = _CORES * _SUBCORES
_CHUNK = 128


def _ln(x, g, b):
    m = jnp.mean(x, axis=-1, keepdims=True)
    v = jnp.mean((x - m) ** 2, axis=-1, keepdims=True)
    return (x - m) / jnp.sqrt(v + 1e-5) * g + b


def _hist_call(src2d, dst2d, n_pad):
    ch = (src2d.shape[0] * src2d.shape[1]) // (_NW * _CHUNK)
    subrows = n_pad // _SUBCORES
    mesh = plsc.VectorSubcoreMesh(core_axis_name="c", subcore_axis_name="s")

    @functools.partial(
        pl.kernel, mesh=mesh,
        out_type=jax.ShapeDtypeStruct((_CORES, n_pad, 128), jnp.float32),
        scratch_types=[
            pltpu.VMEM((8, _CHUNK), jnp.int32),
            pltpu.VMEM((_CHUNK, 128), jnp.float32),
            pltpu.VMEM_SHARED((n_pad, 128), jnp.float32),
        ])
    def k(src_hbm, dst_hbm, out_hbm, idxv, buf, acc):
        c = lax.axis_index("c")
        s = lax.axis_index("s")
        wid = s * _CORES + c
        one16 = jnp.ones((16,), jnp.float32)
        zero16 = jnp.zeros((16,), jnp.float32)

        def fill_zero(t, carry):
            buf[t // 8, pl.ds((t % 8) * 16, 16)] = zero16
            return carry
        lax.fori_loop(0, _CHUNK * 8, fill_zero, 0)

        base = s * subrows

        def zero_acc(t, carry):
            pltpu.sync_copy(buf, acc.at[pl.ds(base + t * _CHUNK, _CHUNK)])
            return carry
        lax.fori_loop(0, subrows // _CHUNK, zero_acc, 0)

        def fill_lo(t, carry):
            buf[t // 4, pl.ds((t % 4) * 16, 16)] = one16
            return carry
        lax.fori_loop(0, _CHUNK * 4, fill_lo, 0)
        plsc.subcore_barrier()

        def outer_src(jj, carry):
            pltpu.sync_copy(src_hbm.at[pl.ds(wid * ch + jj * 8, 8)], idxv)

            def body(j, carry2):
                pltpu.sync_copy(buf, acc.at[idxv.at[j]], add=True)
                return carry2
            lax.fori_loop(0, 8, body, 0)
            return carry
        lax.fori_loop(0, ch // 8, outer_src, 0)

        def fill_swap_lo(t, carry):
            buf[t // 4, pl.ds((t % 4) * 16, 16)] = zero16
            return carry
        lax.fori_loop(0, _CHUNK * 4, fill_swap_lo, 0)

        def fill_swap_hi(t, carry):
            buf[t // 4, pl.ds(64 + (t % 4) * 16, 16)] = one16
            return carry
        lax.fori_loop(0, _CHUNK * 4, fill_swap_hi, 0)

        def outer_dst(jj, carry):
            pltpu.sync_copy(dst_hbm.at[pl.ds(wid * ch + jj * 8, 8)], idxv)

            def body(j, carry2):
                pltpu.sync_copy(buf, acc.at[idxv.at[j]], add=True)
                return carry2
            lax.fori_loop(0, 8, body, 0)
            return carry
        lax.fori_loop(0, ch // 8, outer_dst, 0)
        plsc.subcore_barrier()

        pltpu.sync_copy(acc.at[pl.ds(base, subrows)],
                        out_hbm.at[c, pl.ds(base, subrows)])

    return k(src2d, dst2d)


def _segsum_call(table, src2d, dst2d, n_pad):
    h = table.shape[1]
    ch = (src2d.shape[0] * src2d.shape[1]) // (_NW * _CHUNK)
    subrows = n_pad // _SUBCORES
    mesh = plsc.VectorSubcoreMesh(core_axis_name="c", subcore_axis_name="s")

    @functools.partial(
        pl.kernel, mesh=mesh,
        out_type=jax.ShapeDtypeStruct((_CORES, n_pad, h), jnp.float32),
        scratch_types=[
            pltpu.VMEM((ch // 2, _CHUNK), jnp.int32),
            pltpu.VMEM((ch // 2, _CHUNK), jnp.int32),
            pltpu.VMEM((_CHUNK, h), jnp.float32),
            pltpu.VMEM((_CHUNK, h), jnp.float32),
            pltpu.VMEM_SHARED((n_pad, h), jnp.float32),
            pltpu.SemaphoreType.DMA,
            pltpu.SemaphoreType.DMA,
        ])
    def k(tab_hbm, src_hbm, dst_hbm, out_hbm, sidx, didx, rows_a, rows_b,
          acc, sem_a, sem_b):
        c = lax.axis_index("c")
        s = lax.axis_index("s")
        wid = s * _CORES + c
        zero16 = jnp.zeros((16,), jnp.float32)
        lanes = h // 16

        def fill_zero(t, carry):
            rows_a[t // lanes, pl.ds((t % lanes) * 16, 16)] = zero16
            return carry
        lax.fori_loop(0, _CHUNK * lanes, fill_zero, 0)

        base = s * subrows

        def zero_acc(t, carry):
            pltpu.sync_copy(rows_a, acc.at[pl.ds(base + t * _CHUNK, _CHUNK)])
            return carry
        lax.fori_loop(0, subrows // _CHUNK, zero_acc, 0)
        plsc.subcore_barrier()

        def gat(j, buf, sem):
            return pltpu.make_async_copy(tab_hbm.at[sidx.at[j]], buf, sem)

        def scat(j, buf):
            pltpu.sync_copy(buf, acc.at[didx.at[j]], add=True)

        bch = ch // 2
        for b in range(2):
            pltpu.sync_copy(src_hbm.at[pl.ds(wid * ch + b * bch, bch)], sidx)
            pltpu.sync_copy(dst_hbm.at[pl.ds(wid * ch + b * bch, bch)], didx)
            gat(0, rows_a, sem_a).start()

            def pair(p, carry):
                j = 2 * p
                gat(j + 1, rows_b, sem_b).start()
                gat(j, rows_a, sem_a).wait()
                scat(j, rows_a)
                gat(j + 2, rows_a, sem_a).start()
                gat(j + 1, rows_b, sem_b).wait()
                scat(j + 1, rows_b)
                return carry
            lax.fori_loop(0, bch // 2 - 1, pair, 0)

            je = bch - 2
            gat(je + 1, rows_b, sem_b).start()
            gat(je, rows_a, sem_a).wait()
            scat(je, rows_a)
            gat(je + 1, rows_b, sem_b).wait()
            scat(je + 1, rows_b)
        plsc.subcore_barrier()

        pltpu.sync_copy(acc.at[pl.ds(base, subrows)],
                        out_hbm.at[c, pl.ds(base, subrows)])

    return k(table, src2d, dst2d)


def _stats_call(hist, w1s, n, e, n_pad, h):
    import numpy as _np
    iters = max(int(e).bit_length(), 1)
    kneed = float((n - 1) // 2 + 1)
    mean_c = float(_np.float32(_np.float32(e / n) + _np.float32(1e-6)))

    def body(h_ref, w_ref, sw_ref, aux_ref):
        hsum = h_ref[0] + h_ref[1]
        deg = hsum[:, 0:1]
        indeg = hsum[:, 64:65]
        rows = lax.broadcasted_iota(jnp.int32, (n_pad, 1), 0)
        mask = rows < n

        def bs(i, lohi):
            lo, hi = lohi
            mid = (lo + hi) // 2
            le = jnp.logical_and(mask, deg <= mid.astype(jnp.float32))
            cnt = jnp.sum(jnp.where(le, 1.0, 0.0))
            pred = cnt >= kneed
            return (jnp.where(pred, lo, mid + 1), jnp.where(pred, mid, hi))
        lo, _hi = lax.fori_loop(0, iters, bs, (jnp.int32(0), jnp.int32(e)))
        med = lo.astype(jnp.float32)

        deg = deg[:n]
        indeg = indeg[:n]
        s0 = deg / mean_c
        s1 = jnp.log(1.0 + deg)
        s2 = 1.0 / jnp.sqrt(jnp.maximum(deg, 1.0))
        s3 = (deg > med).astype(jnp.float32)
        w = w_ref[...]
        sw_ref[...] = (s0 * w[0:1, :] + s1 * w[1:2, :]
                       + s2 * w[2:3, :] + s3 * w[3:4, :])
        dis = 1.0 / jnp.sqrt(indeg + 1.0)
        invc = 1.0 / jnp.maximum(indeg, 1.0)
        aux_ref[...] = jnp.concatenate(
            [dis, invc, jnp.zeros((n, h - 2), jnp.float32)], axis=1)

    return pl.pallas_call(
        body,
        out_shape=[jax.ShapeDtypeStruct((n, h), jnp.float32)] * 2,
    )(hist, w1s)


def _bn_for(n):
    for b in (2048, 2000, 1600, 1280, 1000, 800, 512, 400, 200, 104, 8):
        if n % b == 0 and b % 8 == 0:
            return b
    return n


def _node_spec(bn, h):
    return pl.BlockSpec((bn, h), lambda i: (i, 0))


def _full_spec(a):
    return pl.BlockSpec(a.shape, lambda i: tuple(0 for _ in a.shape))


def _dense1_call(x_pad, sw, aux, w1x, b1, g1, be1, w2, b2, wg):
    n, d = x_pad.shape
    h = w2.shape[0]
    bn = _bn_for(n)

    def body(x_ref, sw_ref, aux_ref, w1x_ref, b1_ref, g1_ref, be1_ref,
             w2_ref, b2_ref, wg_ref, h1_ref, y_ref):
        xb = x_ref[...]
        nrm = jnp.sqrt(jnp.sum(xb * xb, axis=-1, keepdims=True))
        xn = xb / jnp.maximum(nrm, 1e-12)
        pre = (jnp.dot(xn, w1x_ref[...], preferred_element_type=jnp.float32)
               + sw_ref[...] + b1_ref[...])
        hh = jnp.maximum(_ln(pre, g1_ref[...], be1_ref[...]), 0.0)
        h1 = jnp.dot(hh, w2_ref[...], preferred_element_type=jnp.float32) + b2_ref[...]
        xw = jnp.dot(h1, wg_ref[...], preferred_element_type=jnp.float32)
        h1_ref[...] = h1
        y_ref[...] = xw * aux_ref[:, 0:1]

    return pl.pallas_call(
        body, grid=(n // bn,),
        in_specs=[_node_spec(bn, d), _node_spec(bn, h), _node_spec(bn, h),
                  _full_spec(w1x), _full_spec(b1), _full_spec(g1),
                  _full_spec(be1), _full_spec(w2), _full_spec(b2),
                  _full_spec(wg)],
        out_specs=[_node_spec(bn, h), _node_spec(bn, h)],
        out_shape=[jax.ShapeDtypeStruct((n, h), jnp.float32)] * 2,
    )(x_pad, sw, aux, w1x, b1, g1, be1, w2, b2, wg)


def _dense2_call(h1, y, p1, aux, bg, gn1, bn1, wsr, bsl):
    n, h = h1.shape
    bn = _bn_for(n)

    def body(h1_ref, y_ref, p_ref, aux_ref, bg_ref, gn1_ref, bn1_ref,
             wsr_ref, bsl_ref, h2_ref, hr_ref):
        ssum = p_ref[0] + p_ref[1] + y_ref[...]
        agg = ssum * aux_ref[:, 0:1] + bg_ref[...]
        hn = jnp.maximum(_ln(agg, gn1_ref[...], bn1_ref[...]), 0.0)
        h2 = h1_ref[...] + hn
        h2_ref[...] = h2
        hr_ref[...] = (jnp.dot(h2, wsr_ref[...], preferred_element_type=jnp.float32)
                       + bsl_ref[...])

    p_spec = pl.BlockSpec((_CORES, bn, h), lambda i: (0, i, 0))
    return pl.pallas_call(
        body, grid=(n // bn,),
        in_specs=[_node_spec(bn, h), _node_spec(bn, h), p_spec,
                  _node_spec(bn, h), _full_spec(bg), _full_spec(gn1),
                  _full_spec(bn1), _full_spec(wsr), _full_spec(bsl)],
        out_specs=[_node_spec(bn, h), _node_spec(bn, h)],
        out_shape=[jax.ShapeDtypeStruct((n, h), jnp.float32)] * 2,
    )(h1, y, p1, aux, bg, gn1, bn1, wsr, bsl)


def _dense3_call(h1, h2, hr, p2, aux, wsl, gn2, bn2, wjk1, wjk2, wjk3, bjk,
                 wc1, bc1, wc2p, bc2p):
    n, h = h1.shape
    bn = _bn_for(n)

    def body(h1_ref, h2_ref, hr_ref, p_ref, aux_ref, wsl_ref, gn2_ref,
             bn2_ref, wjk1_ref, wjk2_ref, wjk3_ref, bjk_ref, wc1_ref,
             bc1_ref, wc2_ref, bc2_ref, out_ref):
        mean = (p_ref[0] + p_ref[1]) * aux_ref[:, 1:2]
        hn = (jnp.dot(mean, wsl_ref[...], preferred_element_type=jnp.float32)
              + hr_ref[...])
        hn = jnp.maximum(_ln(hn, gn2_ref[...], bn2_ref[...]), 0.0)
        h2b = h2_ref[...]
        h3 = h2b + hn
        jk = (jnp.dot(h1_ref[...], wjk1_ref[...], preferred_element_type=jnp.float32)
              + jnp.dot(h2b, wjk2_ref[...], preferred_element_type=jnp.float32)
              + jnp.dot(h3, wjk3_ref[...], preferred_element_type=jnp.float32)
              + bjk_ref[...])
        z = jnp.maximum(
            jnp.dot(jk, wc1_ref[...], preferred_element_type=jnp.float32)
            + bc1_ref[...], 0.0)
        out_ref[...] = (jnp.dot(z, wc2_ref[...], preferred_element_type=jnp.float32)
                        + bc2_ref[...])

    p_spec = pl.BlockSpec((_CORES, bn, h), lambda i: (0, i, 0))
    return pl.pallas_call(
        body, grid=(n // bn,),
        in_specs=[_node_spec(bn, h), _node_spec(bn, h), _node_spec(bn, h),
                  p_spec, _node_spec(bn, h), _full_spec(wsl), _full_spec(gn2),
                  _full_spec(bn2), _full_spec(wjk1), _full_spec(wjk2),
                  _full_spec(wjk3), _full_spec(bjk), _full_spec(wc1),
                  _full_spec(bc1), _full_spec(wc2p), _full_spec(bc2p)],
        out_specs=_node_spec(bn, h),
        out_shape=jax.ShapeDtypeStruct((n, h), jnp.float32),
    )(h1, h2, hr, p2, aux, wsl, gn2, bn2, wjk1, wjk2, wjk3, bjk,
      wc1, bc1, wc2p, bc2p)


def kernel(x, edge_index, W1, b1, g1, be1, W2, b2, Wg, bg, gn1, bn1, Wsl,
           bsl, Wsr, gn2, bn2, Wjk, bjk, Wc1, bc1, Wc2, bc2):
    n, d = x.shape
    h = W2.shape[0]
    c_out = Wc2.shape[1]
    e = edge_index.shape[1]

    n_pad = -(-n // 2560) * 2560
    e_pad = -(-e // (_NW * _CHUNK * 8)) * (_NW * _CHUNK * 8)

    src = edge_index[0]
    dst = edge_index[1]
    pad = e_pad - e
    if pad:
        pad_dst = n + (jnp.arange(pad, dtype=jnp.int32) % (n_pad - n))
        pad_src = jnp.arange(pad, dtype=jnp.int32) % n
        srch = jnp.concatenate([src, pad_dst])
        srcg = jnp.concatenate([src, pad_src])
        dstp = jnp.concatenate([dst, pad_dst])
    else:
        srch, srcg, dstp = src, src, dst
    srch2d = srch.reshape(e_pad // _CHUNK, _CHUNK)
    srcg2d = srcg.reshape(e_pad // _CHUNK, _CHUNK)
    dst2d = dstp.reshape(e_pad // _CHUNK, _CHUNK)

    w1x = W1[:d]
    w1s = W1[d:]
    row = lambda v: v.reshape(1, -1)
    wjk1, wjk2, wjk3 = Wjk[:h], Wjk[h:2 * h], Wjk[2 * h:]
    wc2p = jnp.zeros((h, h), jnp.float32).at[:, :c_out].set(Wc2)
    bc2p = jnp.zeros((1, h), jnp.float32).at[0, :c_out].set(bc2)

    hist = _hist_call(srch2d, dst2d, n_pad)
    sw, aux = _stats_call(hist, w1s, n, e, n_pad, h)
    h1, y = _dense1_call(x, sw, aux, w1x, row(b1), row(g1), row(be1),
                         W2, row(b2), Wg)
    p1 = _segsum_call(y, srcg2d, dst2d, n_pad)
    h2, hr = _dense2_call(h1, y, p1, aux, row(bg), row(gn1), row(bn1),
                          Wsr, row(bsl))
    p2 = _segsum_call(h2, srcg2d, dst2d, n_pad)
    out = _dense3_call(h1, h2, hr, p2, aux, Wsl, row(gn2), row(bn2),
                       wjk1, wjk2, wjk3, row(bjk), Wc1, row(bc1), wc2p, bc2p)
    return out[:, :c_out]
